# Initial kernel scaffold; baseline (speedup 1.0000x reference)
import jax
import jax.numpy as jnp
from jax.experimental import pallas as pl


def kernel(local_conv, local_gate, W, b):
    raise NotImplementedError("write your pallas kernel here")


# speedup vs baseline: 60.5838x; 60.5838x over previous
import jax
import jax.numpy as jnp
from jax.experimental import pallas as pl

_B, _T, _C = 4, 4096, 1024
_TOP_K = 8
_C_BLK = 256

_SORT8 = (
    (0, 1), (2, 3), (4, 5), (6, 7),
    (0, 2), (1, 3), (4, 6), (5, 7),
    (1, 2), (5, 6),
    (0, 4), (1, 5), (2, 6), (3, 7),
    (2, 4), (3, 5),
    (1, 2), (3, 4), (5, 6),
)

_BITONIC8 = (
    (0, 4), (1, 5), (2, 6), (3, 7),
    (0, 2), (1, 3), (4, 6), (5, 7),
    (0, 1), (2, 3), (4, 5), (6, 7),
)


def _compare_exchange(planes, net):
    planes = list(planes)
    for i, j in net:
        hi = jnp.maximum(planes[i], planes[j])
        lo = jnp.minimum(planes[i], planes[j])
        planes[i], planes[j] = hi, lo
    return planes


def _topk_mean_kernel(conv_ref, gate_ref, out_ref):
    x = conv_ref[0] * gate_ref[0]
    r = _T // _TOP_K
    planes = [x[j * r:(j + 1) * r, :] for j in range(_TOP_K)]
    planes = _compare_exchange(planes, _SORT8)
    while r > 1:
        h = r // 2
        a = [p[:h, :] for p in planes]
        b = [p[h:, :] for p in planes]
        planes = [jnp.maximum(a[i], b[7 - i]) for i in range(_TOP_K)]
        planes = _compare_exchange(planes, _BITONIC8)
        r = h
    acc = planes[0]
    for p in planes[1:]:
        acc = acc + p
    out_ref[0, 0, :] = acc[0, :] * (1.0 / _TOP_K)


def _dense_kernel(pooled_ref, w_ref, b_ref, out_ref):
    acc = jnp.dot(pooled_ref[...], w_ref[...],
                  preferred_element_type=jnp.float32)
    out_ref[...] = jnp.maximum(acc + b_ref[...], 0.0)


def kernel(local_conv, local_gate, W, b):
    pooled = pl.pallas_call(
        _topk_mean_kernel,
        grid=(_B, _C // _C_BLK),
        in_specs=[
            pl.BlockSpec((1, _T, _C_BLK), lambda i, j: (i, 0, j)),
            pl.BlockSpec((1, _T, _C_BLK), lambda i, j: (i, 0, j)),
        ],
        out_specs=pl.BlockSpec((1, 1, _C_BLK), lambda i, j: (i, 0, j)),
        out_shape=jax.ShapeDtypeStruct((_B, 1, _C), jnp.float32),
    )(local_conv, local_gate)
    pooled = pooled.reshape(_B, _C)

    out = pl.pallas_call(
        _dense_kernel,
        in_specs=[
            pl.BlockSpec((_B, _C), lambda: (0, 0)),
            pl.BlockSpec((_C, _C), lambda: (0, 0)),
            pl.BlockSpec((_C,), lambda: (0,)),
        ],
        out_specs=pl.BlockSpec((_B, _C), lambda: (0, 0)),
        out_shape=jax.ShapeDtypeStruct((_B, _C), jnp.float32),
    )(pooled, W, b)
    return out


# speedup vs baseline: 78.3071x; 1.2925x over previous
import jax
import jax.numpy as jnp
from jax.experimental import pallas as pl

_B, _T, _C = 4, 4096, 1024
_TOP_K = 8
_C_BLK = 256

_SORT8 = (
    (0, 1), (2, 3), (4, 5), (6, 7),
    (0, 2), (1, 3), (4, 6), (5, 7),
    (1, 2), (5, 6),
    (0, 4), (1, 5), (2, 6), (3, 7),
    (2, 4), (3, 5),
    (1, 2), (3, 4), (5, 6),
)

_BITONIC8 = (
    (0, 4), (1, 5), (2, 6), (3, 7),
    (0, 2), (1, 3), (4, 6), (5, 7),
    (0, 1), (2, 3), (4, 5), (6, 7),
)


def _compare_exchange(planes, net):
    planes = list(planes)
    for i, j in net:
        hi = jnp.maximum(planes[i], planes[j])
        lo = jnp.minimum(planes[i], planes[j])
        planes[i], planes[j] = hi, lo
    return planes


_CHUNK = 64


def _merge_sorted(carry, planes):
    merged = [jnp.maximum(carry[i], planes[7 - i]) for i in range(_TOP_K)]
    return _compare_exchange(merged, _BITONIC8)


def _topk_mean_kernel(conv_ref, gate_ref, out_ref):
    def load_sorted(base):
        planes = [
            conv_ref[0, pl.ds(base + 8 * j, 8), :]
            * gate_ref[0, pl.ds(base + 8 * j, 8), :]
            for j in range(_TOP_K)
        ]
        return _compare_exchange(planes, _SORT8)

    def body(i, carry):
        return tuple(_merge_sorted(list(carry), load_sorted(i * _CHUNK)))

    carry = jax.lax.fori_loop(
        1, _T // _CHUNK, body, tuple(load_sorted(0)), unroll=4)
    planes = list(carry)
    r = 8
    while r > 1:
        h = r // 2
        a = [p[:h, :] for p in planes]
        b = [p[h:, :] for p in planes]
        planes = [jnp.maximum(a[i], b[7 - i]) for i in range(_TOP_K)]
        planes = _compare_exchange(planes, _BITONIC8)
        r = h
    acc = planes[0]
    for p in planes[1:]:
        acc = acc + p
    out_ref[0, 0, :] = acc[0, :] * (1.0 / _TOP_K)


def _dense_kernel(pooled_ref, w_ref, b_ref, out_ref):
    acc = jnp.dot(pooled_ref[...], w_ref[...],
                  preferred_element_type=jnp.float32)
    out_ref[...] = jnp.maximum(acc + b_ref[...], 0.0)


def kernel(local_conv, local_gate, W, b):
    pooled = pl.pallas_call(
        _topk_mean_kernel,
        grid=(_B, _C // _C_BLK),
        in_specs=[
            pl.BlockSpec((1, _T, _C_BLK), lambda i, j: (i, 0, j)),
            pl.BlockSpec((1, _T, _C_BLK), lambda i, j: (i, 0, j)),
        ],
        out_specs=pl.BlockSpec((1, 1, _C_BLK), lambda i, j: (i, 0, j)),
        out_shape=jax.ShapeDtypeStruct((_B, 1, _C), jnp.float32),
    )(local_conv, local_gate)
    pooled = pooled.reshape(_B, _C)

    out = pl.pallas_call(
        _dense_kernel,
        in_specs=[
            pl.BlockSpec((_B, _C), lambda: (0, 0)),
            pl.BlockSpec((_C, _C), lambda: (0, 0)),
            pl.BlockSpec((_C,), lambda: (0,)),
        ],
        out_specs=pl.BlockSpec((_B, _C), lambda: (0, 0)),
        out_shape=jax.ShapeDtypeStruct((_B, _C), jnp.float32),
    )(pooled, W, b)
    return out


# speedup vs baseline: 80.6705x; 1.0302x over previous
import jax
import jax.numpy as jnp
from jax.experimental import pallas as pl

_B, _T, _C = 4, 4096, 1024
_TOP_K = 8
_C_BLK = 256

_SORT8 = (
    (0, 1), (2, 3), (4, 5), (6, 7),
    (0, 2), (1, 3), (4, 6), (5, 7),
    (1, 2), (5, 6),
    (0, 4), (1, 5), (2, 6), (3, 7),
    (2, 4), (3, 5),
    (1, 2), (3, 4), (5, 6),
)

_BITONIC8 = (
    (0, 4), (1, 5), (2, 6), (3, 7),
    (0, 2), (1, 3), (4, 6), (5, 7),
    (0, 1), (2, 3), (4, 5), (6, 7),
)


def _compare_exchange(planes, net):
    planes = list(planes)
    for i, j in net:
        hi = jnp.maximum(planes[i], planes[j])
        lo = jnp.minimum(planes[i], planes[j])
        planes[i], planes[j] = hi, lo
    return planes


_CHUNK = 64


def _merge_sorted(carry, planes):
    merged = [jnp.maximum(carry[i], planes[7 - i]) for i in range(_TOP_K)]
    return _compare_exchange(merged, _BITONIC8)


def _topk_mean_kernel(conv_ref, gate_ref, out_ref):
    def load_sorted(base):
        planes = [
            conv_ref[0, base + 8 * j:base + 8 * (j + 1), :]
            * gate_ref[0, base + 8 * j:base + 8 * (j + 1), :]
            for j in range(_TOP_K)
        ]
        return _compare_exchange(planes, _SORT8)

    planes = load_sorted(0)
    for i in range(1, _T // _CHUNK):
        planes = _merge_sorted(planes, load_sorted(i * _CHUNK))
    r = 8
    while r > 1:
        h = r // 2
        a = [p[:h, :] for p in planes]
        b = [p[h:, :] for p in planes]
        planes = [jnp.maximum(a[i], b[7 - i]) for i in range(_TOP_K)]
        planes = _compare_exchange(planes, _BITONIC8)
        r = h
    acc = planes[0]
    for p in planes[1:]:
        acc = acc + p
    out_ref[0, 0, :] = acc[0, :] * (1.0 / _TOP_K)


def _dense_kernel(pooled_ref, w_ref, b_ref, out_ref):
    acc = jnp.dot(pooled_ref[...], w_ref[...],
                  preferred_element_type=jnp.float32)
    out_ref[...] = jnp.maximum(acc + b_ref[...], 0.0)


def kernel(local_conv, local_gate, W, b):
    pooled = pl.pallas_call(
        _topk_mean_kernel,
        grid=(_B, _C // _C_BLK),
        in_specs=[
            pl.BlockSpec((1, _T, _C_BLK), lambda i, j: (i, 0, j)),
            pl.BlockSpec((1, _T, _C_BLK), lambda i, j: (i, 0, j)),
        ],
        out_specs=pl.BlockSpec((1, 1, _C_BLK), lambda i, j: (i, 0, j)),
        out_shape=jax.ShapeDtypeStruct((_B, 1, _C), jnp.float32),
    )(local_conv, local_gate)
    pooled = pooled.reshape(_B, _C)

    out = pl.pallas_call(
        _dense_kernel,
        in_specs=[
            pl.BlockSpec((_B, _C), lambda: (0, 0)),
            pl.BlockSpec((_C, _C), lambda: (0, 0)),
            pl.BlockSpec((_C,), lambda: (0,)),
        ],
        out_specs=pl.BlockSpec((_B, _C), lambda: (0, 0)),
        out_shape=jax.ShapeDtypeStruct((_B, _C), jnp.float32),
    )(pooled, W, b)
    return out
